# baseline (speedup 1.0000x reference)
import functools

import jax
import jax.numpy as jnp
from jax import lax
from jax.experimental import pallas as pl
from jax.experimental.pallas import tpu as pltpu
from jax.experimental.pallas import tpu_sc as plsc

_BATCH = 16384
_DIM = 32
_NC = 2
_NS = 16
_NW = _NC * _NS
_ROWS_PER_W = _BATCH // _NW
_HALF = 8
_NHALF = _ROWS_PER_W // _HALF
_IDXBUF = _ROWS_PER_W + 32

_mesh = plsc.VectorSubcoreMesh(core_axis_name="c", subcore_axis_name="s")

_scratch = (
    [pltpu.VMEM((_IDXBUF,), jnp.int32)]
    + [pltpu.VMEM((_HALF, _DIM), jnp.float32) for _ in range(2)]
    + [pltpu.VMEM((_DIM, 128), jnp.float32) for _ in range(2 * _HALF)]
    + [pltpu.SemaphoreType.DMA, pltpu.SemaphoreType.DMA,
       pltpu.SemaphoreType.DMA, pltpu.SemaphoreType.DMA]
)


@functools.partial(
    pl.kernel,
    mesh=_mesh,
    out_type=jax.ShapeDtypeStruct((_BATCH, _DIM), jnp.float32),
    scratch_types=_scratch,
    compiler_params=pltpu.CompilerParams(needs_layout_passes=False),
)
def _gather_kernel(idx_hbm, table_t_hbm, out_hbm, idx_v, half_a, half_b,
                   *rest):
    halves = (half_a, half_b)
    bufs = (rest[:_HALF], rest[_HALF:2 * _HALF])
    gsems = rest[2 * _HALF:2 * _HALF + 2]
    osems = rest[2 * _HALF + 2:]
    wid = lax.axis_index("s") * _NC + lax.axis_index("c")
    base = wid * _ROWS_PER_W
    pltpu.sync_copy(
        idx_hbm.at[pl.ds(base, _ROWS_PER_W)], idx_v.at[pl.ds(0, _ROWS_PER_W)]
    )
    zeros = jnp.zeros((16,), jnp.int32)
    idx_v[pl.ds(_ROWS_PER_W, 16)] = zeros
    idx_v[pl.ds(_ROWS_PER_W + 16, 16)] = zeros

    lanes_lo = lax.iota(jnp.int32, 16)
    lanes_hi = lanes_lo + 16

    def fire_half(h, p):
        v = idx_v[pl.ds(h * _HALF, 16)]
        for k in range(_HALF):
            q = pl.multiple_of(
                lax.shift_left(lax.shift_right_logical(v[k], 7), 7), 128
            )
            pltpu.async_copy(
                table_t_hbm.at[:, pl.ds(q, 128)], bufs[p][k], gsems[p]
            )

    def drain_half(p):
        for k in range(_HALF):
            pltpu.make_async_copy(
                table_t_hbm.at[:, pl.ds(0, 128)], bufs[p][k], gsems[p]
            ).wait()

    def extract_and_emit(h, p):
        v = idx_v[pl.ds(h * _HALF, 16)]
        for k in range(_HALF):
            r = jnp.full((16,), lax.bitwise_and(v[k], 127), jnp.int32)
            j = jnp.full((16,), k, jnp.int32)
            lo = plsc.load_gather(bufs[p][k], [lanes_lo, r])
            plsc.store_scatter(halves[p], [j, lanes_lo], lo)
            hi = plsc.load_gather(bufs[p][k], [lanes_hi, r])
            plsc.store_scatter(halves[p], [j, lanes_hi], hi)
        pltpu.async_copy(
            halves[p], out_hbm.at[pl.ds(base + h * _HALF, _HALF)], osems[p]
        )

    def drain_out(p):
        pltpu.make_async_copy(
            halves[p], out_hbm.at[pl.ds(base, _HALF)], osems[p]
        ).wait()

    fire_half(0, 0)
    fire_half(1, 1)

    def body(g, started):
        for p in range(2):
            h = 2 * g + p
            drain_half(p)
            @pl.when(started[p] != 0)
            def _():
                drain_out(p)
            extract_and_emit(h, p)
            fire_half(h + 2, p)
        return (jnp.int32(1), jnp.int32(1))

    lax.fori_loop(0, _NHALF // 2, body, (jnp.int32(0), jnp.int32(0)))
    drain_half(0)
    drain_half(1)
    drain_out(0)
    drain_out(1)


def kernel(y_true, proxies):
    return _gather_kernel(y_true.astype(jnp.int32), proxies.T)


# speedup vs baseline: 1.0042x; 1.0042x over previous
import functools

import jax
import jax.numpy as jnp
from jax import lax
from jax.experimental import pallas as pl
from jax.experimental.pallas import tpu as pltpu
from jax.experimental.pallas import tpu_sc as plsc

_BATCH = 16384
_DIM = 32
_NC = 2
_NS = 16
_NW = _NC * _NS
_ROWS_PER_W = _BATCH // _NW
_HALF = 8
_NHALF = _ROWS_PER_W // _HALF
_IDXBUF = _ROWS_PER_W + 32

_mesh = plsc.VectorSubcoreMesh(core_axis_name="c", subcore_axis_name="s")

_scratch = (
    [pltpu.VMEM((_IDXBUF,), jnp.int32)]
    + [pltpu.VMEM((_HALF, _DIM), jnp.float32) for _ in range(2)]
    + [pltpu.VMEM((_DIM, 128), jnp.float32) for _ in range(2 * _HALF)]
    + [pltpu.SemaphoreType.DMA, pltpu.SemaphoreType.DMA,
       pltpu.SemaphoreType.DMA, pltpu.SemaphoreType.DMA]
)


@functools.partial(
    pl.kernel,
    mesh=_mesh,
    out_type=jax.ShapeDtypeStruct((_BATCH, _DIM), jnp.float32),
    scratch_types=_scratch,
    compiler_params=pltpu.CompilerParams(needs_layout_passes=False),
)
def _gather_kernel(idx_hbm, table_t_hbm, out_hbm, idx_v, half_a, half_b,
                   *rest):
    halves = (half_a, half_b)
    bufs = (rest[:_HALF], rest[_HALF:2 * _HALF])
    gsems = rest[2 * _HALF:2 * _HALF + 2]
    osems = rest[2 * _HALF + 2:]
    wid = lax.axis_index("s") * _NC + lax.axis_index("c")
    base = wid * _ROWS_PER_W
    pltpu.sync_copy(
        idx_hbm.at[pl.ds(base, _ROWS_PER_W)], idx_v.at[pl.ds(0, _ROWS_PER_W)]
    )
    zeros = jnp.zeros((16,), jnp.int32)
    idx_v[pl.ds(_ROWS_PER_W, 16)] = zeros
    idx_v[pl.ds(_ROWS_PER_W + 16, 16)] = zeros

    lanes_lo = lax.iota(jnp.int32, 16)
    lanes_hi = lanes_lo + 16

    def fire_half(h, p):
        v = idx_v[pl.ds(h * _HALF, 16)]
        for k in range(_HALF):
            q = pl.multiple_of(
                lax.shift_left(lax.shift_right_logical(v[k], 7), 7), 128
            )
            for g in range(_DIM // 8):
                pltpu.async_copy(
                    table_t_hbm.at[pl.ds(g * 8, 8), pl.ds(q, 128)],
                    bufs[p][k].at[pl.ds(g * 8, 8), :],
                    gsems[p],
                )

    def drain_half(p):
        for k in range(_HALF):
            for g in range(_DIM // 8):
                pltpu.make_async_copy(
                    table_t_hbm.at[pl.ds(g * 8, 8), pl.ds(0, 128)],
                    bufs[p][k].at[pl.ds(g * 8, 8), :],
                    gsems[p],
                ).wait()

    def extract_and_emit(h, p):
        v = idx_v[pl.ds(h * _HALF, 16)]
        for k in range(_HALF):
            r = jnp.full((16,), lax.bitwise_and(v[k], 127), jnp.int32)
            j = jnp.full((16,), k, jnp.int32)
            lo = plsc.load_gather(bufs[p][k], [lanes_lo, r])
            plsc.store_scatter(halves[p], [j, lanes_lo], lo)
            hi = plsc.load_gather(bufs[p][k], [lanes_hi, r])
            plsc.store_scatter(halves[p], [j, lanes_hi], hi)
        pltpu.async_copy(
            halves[p], out_hbm.at[pl.ds(base + h * _HALF, _HALF)], osems[p]
        )

    def drain_out(p):
        pltpu.make_async_copy(
            halves[p], out_hbm.at[pl.ds(base, _HALF)], osems[p]
        ).wait()

    fire_half(0, 0)
    fire_half(1, 1)

    def body(g, started):
        for p in range(2):
            h = 2 * g + p
            drain_half(p)
            @pl.when(started[p] != 0)
            def _():
                drain_out(p)
            extract_and_emit(h, p)
            fire_half(h + 2, p)
        return (jnp.int32(1), jnp.int32(1))

    lax.fori_loop(0, _NHALF // 2, body, (jnp.int32(0), jnp.int32(0)))
    drain_half(0)
    drain_half(1)
    drain_out(0)
    drain_out(1)


def kernel(y_true, proxies):
    return _gather_kernel(y_true.astype(jnp.int32), proxies.T)
